# Initial kernel scaffold; baseline (speedup 1.0000x reference)
import jax
import jax.numpy as jnp
from jax.experimental import pallas as pl


def kernel(input_tensor, embed, Wr, br, W1, b1, W2, b2, Wlm, blm):
    raise NotImplementedError("write your pallas kernel here")


# speedup vs baseline: 1.8853x; 1.8853x over previous
import functools

import jax
import jax.numpy as jnp
from jax import lax
from jax.experimental import pallas as pl
from jax.experimental.pallas import tpu as pltpu
from jax.experimental.pallas import tpu_sc as plsc

E = 16
TOP_K = 2
D = 1024
F = 2048
V = 64
T = 2048
C = 384
TEMP = 2.0

NC = 2
NS = 16
NW = NC * NS


def _make_sc_gather(n_rows_table, d, b, rows_per_chunk):
    del n_rows_table
    b_per_w = b // NW
    assert b % (8 * NW) == 0
    assert b_per_w % rows_per_chunk == 0
    n_chunks = b_per_w // rows_per_chunk
    mesh = plsc.VectorSubcoreMesh(core_axis_name="c", subcore_axis_name="s")

    @functools.partial(
        pl.kernel,
        mesh=mesh,
        out_type=jax.ShapeDtypeStruct((b, d), jnp.float32),
        scratch_types=[
            pltpu.VMEM((rows_per_chunk,), jnp.int32),
            pltpu.VMEM((rows_per_chunk, d), jnp.float32),
            pltpu.SemaphoreType.DMA,
        ],
    )
    def gather_kernel(table_hbm, idx_hbm, out_hbm, idx_v, rows_v, sem):
        wid = lax.axis_index("s") * NC + lax.axis_index("c")
        base = wid * b_per_w
        for j in range(n_chunks):
            off = base + j * rows_per_chunk
            pltpu.sync_copy(idx_hbm.at[pl.ds(off, rows_per_chunk)], idx_v)
            pltpu.async_copy(table_hbm.at[idx_v], rows_v, sem).wait()
            pltpu.sync_copy(rows_v, out_hbm.at[pl.ds(off, rows_per_chunk)])

    return gather_kernel


def _router_body(x_ref, wr_ref, br_ref, sel_ref, gsel_ref, cidx_ref,
                 cmask_ref, m_ref, g_ref, p_ref):
    x = x_ref[...]
    logits = jnp.dot(x, wr_ref[...], preferred_element_type=jnp.float32)
    logits = logits + br_ref[...]
    z = logits * (1.0 / TEMP)
    z = z - jnp.max(z, axis=-1, keepdims=True)
    ez = jnp.exp(z)
    probs = ez / jnp.sum(ez, axis=-1, keepdims=True)

    iota_e = lax.broadcasted_iota(jnp.int32, (1, E), 1).astype(jnp.float32)
    v1 = jnp.max(probs, axis=-1, keepdims=True)
    i1 = jnp.min(jnp.where(probs == v1, iota_e, 1e9), axis=-1, keepdims=True)
    oh1 = (iota_e == i1).astype(jnp.float32)
    probs2 = jnp.where(oh1 > 0, -1.0, probs)
    v2 = jnp.max(probs2, axis=-1, keepdims=True)
    i2 = jnp.min(jnp.where(probs2 == v2, iota_e, 1e9), axis=-1, keepdims=True)
    oh2 = (iota_e == i2).astype(jnp.float32)

    s = v1 + v2
    g1 = v1 / s
    g2 = v2 / s
    m_ref[...] = oh1 + oh2
    g_ref[...] = g1 * oh1 + g2 * oh2

    R = 256
    rows = lax.broadcasted_iota(jnp.int32, (R, R), 0)
    cols = lax.broadcasted_iota(jnp.int32, (R, R), 1)
    tri = (rows > cols).astype(jnp.float32)

    def chunk_body(c, base):
        off = pl.multiple_of(c * R, R)
        mc = m_ref[pl.ds(off, R), :]
        p_ref[pl.ds(off, R), :] = (
            jnp.dot(tri, mc, preferred_element_type=jnp.float32) + base
        )
        return base + jnp.sum(mc, axis=0, keepdims=True)

    lax.fori_loop(0, T // R, chunk_body, jnp.zeros((1, E), jnp.float32))

    iota_t = lax.broadcasted_iota(jnp.int32, (T, 1), 0).astype(jnp.float32)
    qcol = jnp.floor(iota_t * 0.125)
    rcol = iota_t - 8.0 * qcol
    iota_c = lax.broadcasted_iota(jnp.int32, (1, C), 1).astype(jnp.float32)
    for e in range(E):
        pcol = p_ref[:, e : e + 1]
        mcol = m_ref[:, e : e + 1]
        gcol = g_ref[:, e : e + 1]
        key = jnp.where(mcol > 0, pcol, jnp.float32(C))
        oh = (key == iota_c).astype(jnp.float32)
        a = jnp.concatenate([qcol, rcol, gcol], axis=1)
        res = lax.dot_general(
            oh, a, (((0,), (0,)), ((), ())),
            preferred_element_type=jnp.float32,
        )
        sel_ref[:, e : e + 1] = (
            8.0 * res[:, 0:1] + res[:, 1:2]
        ).astype(jnp.int32)
        gsel_ref[:, e : e + 1] = res[:, 2:3]

    p = p_ref[...]
    ps1 = jnp.sum(jnp.where(oh1 > 0, p, 0.0), axis=-1, keepdims=True)
    ps2 = jnp.sum(jnp.where(oh2 > 0, p, 0.0), axis=-1, keepdims=True)
    ok1 = ps1 < C
    ok2 = ps2 < C
    cidx_ref[:, 0:1] = jnp.where(ok1, i1 * C + ps1, 0.0).astype(jnp.int32)
    cidx_ref[:, 1:2] = jnp.where(ok2, i2 * C + ps2, 0.0).astype(jnp.int32)
    cmask_ref[:, 0:1] = ok1.astype(jnp.float32)
    cmask_ref[:, 1:2] = ok2.astype(jnp.float32)


def _router(x, wr, br2):
    return pl.pallas_call(
        _router_body,
        out_shape=(
            jax.ShapeDtypeStruct((C, E), jnp.int32),
            jax.ShapeDtypeStruct((C, E), jnp.float32),
            jax.ShapeDtypeStruct((T, 2), jnp.int32),
            jax.ShapeDtypeStruct((T, 2), jnp.float32),
        ),
        scratch_shapes=[
            pltpu.VMEM((T, E), jnp.float32),
            pltpu.VMEM((T, E), jnp.float32),
            pltpu.VMEM((T, E), jnp.float32),
        ],
    )(x, wr, br2)


FT = 1024
NF = F // FT


def _ffn_body(xg_ref, w1_ref, b1_ref, w2_ref, b2_ref, g_ref, out_ref):
    f = pl.program_id(1)
    h = jnp.dot(xg_ref[...], w1_ref[0], preferred_element_type=jnp.float32)
    h = jax.nn.gelu(h + b1_ref[0])
    y = jnp.dot(h, w2_ref[0], preferred_element_type=jnp.float32)

    @pl.when(f == 0)
    def _init():
        out_ref[...] = y

    @pl.when(f > 0)
    def _acc():
        out_ref[...] = out_ref[...] + y

    @pl.when(f == NF - 1)
    def _fin():
        out_ref[...] = (out_ref[...] + b2_ref[0]) * g_ref[...]


def _ffn(xg, w1, b1, w2, b2, gcolv):
    return pl.pallas_call(
        _ffn_body,
        grid=(E, NF),
        in_specs=[
            pl.BlockSpec((C, D), lambda e, f: (e, 0)),
            pl.BlockSpec((1, D, FT), lambda e, f: (e, 0, f)),
            pl.BlockSpec((1, 1, FT), lambda e, f: (e, 0, f)),
            pl.BlockSpec((1, FT, D), lambda e, f: (e, f, 0)),
            pl.BlockSpec((1, 1, D), lambda e, f: (e, 0, 0)),
            pl.BlockSpec((C, 1), lambda e, f: (e, 0)),
        ],
        out_specs=pl.BlockSpec((C, D), lambda e, f: (e, 0)),
        out_shape=jax.ShapeDtypeStruct((E * C, D), jnp.float32),
    )(xg, w1, b1.reshape(E, 1, F), w2, b2.reshape(E, 1, D), gcolv)


RT = 512
NR = T // RT


def _head_body(x_ref, y0_ref, y1_ref, cm_ref, wlm_ref, blm_ref, out_ref):
    cm = cm_ref[...]
    cog = x_ref[...] + y0_ref[...] * cm[:, 0:1] + y1_ref[...] * cm[:, 1:2]
    out_ref[...] = (
        jnp.dot(cog, wlm_ref[...], preferred_element_type=jnp.float32)
        + blm_ref[...]
    )


def _head(x, y01, cmask, wlm, blm2):
    return pl.pallas_call(
        _head_body,
        grid=(NR,),
        in_specs=[
            pl.BlockSpec((RT, D), lambda r: (r, 0)),
            pl.BlockSpec((RT, D), lambda r: (r, 0)),
            pl.BlockSpec((RT, D), lambda r: (r + NR, 0)),
            pl.BlockSpec((RT, 2), lambda r: (r, 0)),
            pl.BlockSpec((D, V), lambda r: (0, 0)),
            pl.BlockSpec((1, V), lambda r: (0, 0)),
        ],
        out_specs=pl.BlockSpec((RT, V), lambda r: (r, 0)),
        out_shape=jax.ShapeDtypeStruct((T, V), jnp.float32),
    )(x, y01, y01, cmask, wlm, blm2)


def _lazy(maker):
    cache = []

    def call(table, idx):
        if not cache:
            cache.append(maker())
        return cache[0](table, idx)

    return call


_embed_gather = _lazy(lambda: _make_sc_gather(V, D, T, 64))
_dispatch_gather = _lazy(lambda: _make_sc_gather(T, D, E * C, 96))
_combine_gather = _lazy(lambda: _make_sc_gather(E * C, D, 2 * T, 64))


def kernel(input_tensor, embed, Wr, br, W1, b1, W2, b2, Wlm, blm):
    idx = input_tensor.astype(jnp.int32)
    x = _embed_gather(embed, idx)
    sel, gsel, cidx, cmask = _router(x, Wr, br.reshape(1, E))
    xg = _dispatch_gather(x, sel.T.reshape(E * C))
    yg = _ffn(xg, W1, b1, W2, b2, gsel.T.reshape(E * C, 1))
    y01 = _combine_gather(yg, cidx.T.reshape(2 * T))
    return _head(x, y01, cmask, Wlm, blm.reshape(1, V))


# speedup vs baseline: 2.0064x; 1.0642x over previous
import functools

import jax
import jax.numpy as jnp
from jax import lax
from jax.experimental import pallas as pl
from jax.experimental.pallas import tpu as pltpu
from jax.experimental.pallas import tpu_sc as plsc

E = 16
TOP_K = 2
D = 1024
F = 2048
V = 64
VP = 128
T = 2048
C = 384
TEMP = 2.0

NC = 2
NS = 16
NW = NC * NS


def _make_sc_gather(d, b, rows_per_chunk):
    b_per_w = b // NW
    assert b % (8 * NW) == 0
    assert b_per_w % rows_per_chunk == 0
    n_chunks = b_per_w // rows_per_chunk
    mesh = plsc.VectorSubcoreMesh(core_axis_name="c", subcore_axis_name="s")

    @functools.partial(
        pl.kernel,
        mesh=mesh,
        out_type=jax.ShapeDtypeStruct((b, d), jnp.float32),
        scratch_types=[
            pltpu.VMEM((b_per_w,), jnp.int32),
            pltpu.VMEM((rows_per_chunk, d), jnp.float32),
            pltpu.VMEM((rows_per_chunk, d), jnp.float32),
            pltpu.SemaphoreType.DMA,
            pltpu.SemaphoreType.DMA,
            pltpu.SemaphoreType.DMA,
        ],
    )
    def gather_kernel(table_hbm, idx_hbm, out_hbm, idx_v, buf0, buf1,
                      gsem, wsem0, wsem1):
        wid = lax.axis_index("s") * NC + lax.axis_index("c")
        base = wid * b_per_w
        pltpu.sync_copy(idx_hbm.at[pl.ds(base, b_per_w)], idx_v)
        bufs = (buf0, buf1)
        wsems = (wsem0, wsem1)
        pending = [None, None]
        for j in range(n_chunks):
            k = j % 2
            if pending[k] is not None:
                pending[k].wait()
            off = j * rows_per_chunk
            pltpu.async_copy(
                table_hbm.at[idx_v.at[pl.ds(off, rows_per_chunk)]],
                bufs[k], gsem,
            ).wait()
            pending[k] = pltpu.async_copy(
                bufs[k], out_hbm.at[pl.ds(base + off, rows_per_chunk)],
                wsems[k],
            )
        for k in range(2):
            if pending[k] is not None:
                pending[k].wait()

    return gather_kernel


def _router_body(x_ref, wr_ref, br_ref, wlm_ref, blm_ref,
                 sel_ref, gsel_ref, cidx_ref, cmask_ref, xlm_ref,
                 m_ref, g_ref, p_ref):
    x = x_ref[...]
    xlm_ref[...] = (
        jnp.dot(x, wlm_ref[...], preferred_element_type=jnp.float32)
        + blm_ref[...]
    )
    logits = jnp.dot(x, wr_ref[...], preferred_element_type=jnp.float32)
    logits = logits + br_ref[...]
    z = logits * (1.0 / TEMP)
    z = z - jnp.max(z, axis=-1, keepdims=True)
    ez = jnp.exp(z)
    probs = ez / jnp.sum(ez, axis=-1, keepdims=True)

    iota_e = lax.broadcasted_iota(jnp.int32, (1, E), 1).astype(jnp.float32)
    v1 = jnp.max(probs, axis=-1, keepdims=True)
    i1 = jnp.min(jnp.where(probs == v1, iota_e, 1e9), axis=-1, keepdims=True)
    oh1 = (iota_e == i1).astype(jnp.float32)
    probs2 = jnp.where(oh1 > 0, -1.0, probs)
    v2 = jnp.max(probs2, axis=-1, keepdims=True)
    i2 = jnp.min(jnp.where(probs2 == v2, iota_e, 1e9), axis=-1, keepdims=True)
    oh2 = (iota_e == i2).astype(jnp.float32)

    s = v1 + v2
    g1 = v1 / s
    g2 = v2 / s
    m_ref[...] = oh1 + oh2
    g_ref[...] = g1 * oh1 + g2 * oh2

    R = 256
    rows = lax.broadcasted_iota(jnp.int32, (R, R), 0)
    cols = lax.broadcasted_iota(jnp.int32, (R, R), 1)
    tri = (rows > cols).astype(jnp.float32)

    def chunk_body(c, base):
        off = pl.multiple_of(c * R, R)
        mc = m_ref[pl.ds(off, R), :]
        p_ref[pl.ds(off, R), :] = (
            jnp.dot(tri, mc, preferred_element_type=jnp.float32) + base
        )
        return base + jnp.sum(mc, axis=0, keepdims=True)

    lax.fori_loop(0, T // R, chunk_body, jnp.zeros((1, E), jnp.float32))

    iota_t = lax.broadcasted_iota(jnp.int32, (T, 1), 0).astype(jnp.float32)
    qcol = jnp.floor(iota_t * 0.125)
    rcol = iota_t - 8.0 * qcol
    iota_c = lax.broadcasted_iota(jnp.int32, (1, C), 1).astype(jnp.float32)
    for e in range(E):
        pcol = p_ref[:, e : e + 1]
        mcol = m_ref[:, e : e + 1]
        gcol = g_ref[:, e : e + 1]
        key = jnp.where(mcol > 0, pcol, jnp.float32(C))
        oh = (key == iota_c).astype(jnp.float32)
        a = jnp.concatenate([qcol, rcol, gcol], axis=1)
        res = lax.dot_general(
            oh, a, (((0,), (0,)), ((), ())),
            preferred_element_type=jnp.float32,
        )
        sel_ref[:, e : e + 1] = (
            8.0 * res[:, 0:1] + res[:, 1:2]
        ).astype(jnp.int32)
        gsel_ref[:, e : e + 1] = res[:, 2:3]

    p = p_ref[...]
    ps1 = jnp.sum(jnp.where(oh1 > 0, p, 0.0), axis=-1, keepdims=True)
    ps2 = jnp.sum(jnp.where(oh2 > 0, p, 0.0), axis=-1, keepdims=True)
    ok1 = ps1 < C
    ok2 = ps2 < C
    cidx_ref[:, 0:1] = jnp.where(ok1, i1 * C + ps1, 0.0).astype(jnp.int32)
    cidx_ref[:, 1:2] = jnp.where(ok2, i2 * C + ps2, 0.0).astype(jnp.int32)
    cmask_ref[:, 0:1] = ok1.astype(jnp.float32)
    cmask_ref[:, 1:2] = ok2.astype(jnp.float32)


def _router(x, wr, br2, wlm, blm2):
    return pl.pallas_call(
        _router_body,
        out_shape=(
            jax.ShapeDtypeStruct((C, E), jnp.int32),
            jax.ShapeDtypeStruct((C, E), jnp.float32),
            jax.ShapeDtypeStruct((T, 2), jnp.int32),
            jax.ShapeDtypeStruct((T, 2), jnp.float32),
            jax.ShapeDtypeStruct((T, V), jnp.float32),
        ),
        scratch_shapes=[
            pltpu.VMEM((T, E), jnp.float32),
            pltpu.VMEM((T, E), jnp.float32),
            pltpu.VMEM((T, E), jnp.float32),
        ],
    )(x, wr, br2, wlm, blm2)


FT = 1024
NF = F // FT


def _ffn_body(xg_ref, w1_ref, b1_ref, w2_ref, b2_ref, g_ref, wlm_ref,
              zg_ref, acc_ref):
    f = pl.program_id(1)
    h = jnp.dot(xg_ref[...], w1_ref[0], preferred_element_type=jnp.float32)
    h = jax.nn.gelu(h + b1_ref[0])
    y = jnp.dot(h, w2_ref[0], preferred_element_type=jnp.float32)

    @pl.when(f == 0)
    def _init():
        acc_ref[...] = y

    @pl.when(f > 0)
    def _acc():
        acc_ref[...] = acc_ref[...] + y

    @pl.when(f == NF - 1)
    def _fin():
        ye = (acc_ref[...] + b2_ref[0]) * g_ref[...]
        z = jnp.dot(ye, wlm_ref[...], preferred_element_type=jnp.float32)
        zg_ref[...] = jnp.concatenate(
            [z, jnp.zeros((C, VP - V), jnp.float32)], axis=1
        )


def _ffn(xg, w1, b1, w2, b2, gcolv, wlm):
    return pl.pallas_call(
        _ffn_body,
        grid=(E, NF),
        in_specs=[
            pl.BlockSpec((C, D), lambda e, f: (e, 0)),
            pl.BlockSpec((1, D, FT), lambda e, f: (e, 0, f)),
            pl.BlockSpec((1, 1, FT), lambda e, f: (e, 0, f)),
            pl.BlockSpec((1, FT, D), lambda e, f: (e, f, 0)),
            pl.BlockSpec((1, 1, D), lambda e, f: (e, 0, 0)),
            pl.BlockSpec((C, 1), lambda e, f: (e, 0)),
            pl.BlockSpec((D, V), lambda e, f: (0, 0)),
        ],
        out_specs=pl.BlockSpec((C, VP), lambda e, f: (e, 0)),
        out_shape=jax.ShapeDtypeStruct((E * C, VP), jnp.float32),
        scratch_shapes=[pltpu.VMEM((C, D), jnp.float32)],
    )(xg, w1, b1.reshape(E, 1, F), w2, b2.reshape(E, 1, D), gcolv, wlm)


def _combine_body(xlm_ref, z0_ref, z1_ref, cm_ref, out_ref):
    cm = cm_ref[...]
    out_ref[...] = (
        xlm_ref[...]
        + z0_ref[:, :V] * cm[:, 0:1]
        + z1_ref[:, :V] * cm[:, 1:2]
    )


def _combine(xlm, z01, cmask):
    return pl.pallas_call(
        _combine_body,
        grid=(1,),
        in_specs=[
            pl.BlockSpec((T, V), lambda r: (0, 0)),
            pl.BlockSpec((T, VP), lambda r: (0, 0)),
            pl.BlockSpec((T, VP), lambda r: (1, 0)),
            pl.BlockSpec((T, 2), lambda r: (0, 0)),
        ],
        out_specs=pl.BlockSpec((T, V), lambda r: (0, 0)),
        out_shape=jax.ShapeDtypeStruct((T, V), jnp.float32),
    )(xlm, z01, z01, cmask)


def _lazy(maker):
    cache = []

    def call(table, idx):
        if not cache:
            cache.append(maker())
        return cache[0](table, idx)

    return call


_embed_gather = _lazy(lambda: _make_sc_gather(D, T, 32))
_dispatch_gather = _lazy(lambda: _make_sc_gather(D, E * C, 48))
_combine_gather = _lazy(lambda: _make_sc_gather(VP, 2 * T, 64))


def kernel(input_tensor, embed, Wr, br, W1, b1, W2, b2, Wlm, blm):
    idx = input_tensor.astype(jnp.int32)
    x = _embed_gather(embed, idx)
    sel, gsel, cidx, cmask, xlm = _router(
        x, Wr, br.reshape(1, E), Wlm, blm.reshape(1, V)
    )
    xg = _dispatch_gather(x, sel.T.reshape(E * C))
    zg = _ffn(xg, W1, b1, W2, b2, gsel.T.reshape(E * C, 1), Wlm)
    z01 = _combine_gather(zg, cidx.T.reshape(2 * T))
    return _combine(xlm, z01, cmask)
